# Initial kernel scaffold; baseline (speedup 1.0000x reference)
import functools

import jax
import jax.numpy as jnp
from jax import lax
from jax.experimental import pallas as pl
from jax.experimental.pallas import tpu as pltpu
from jax.experimental.pallas import tpu_sc as plsc

N_NODES = 10000
N_EDGES = 160000
D = 256
ED = 16
HALF = D // 2

NC = 2
NS = 16
EDGES_PER_SUB = N_EDGES // NS
EB = 80
N_BLOCKS = EDGES_PER_SUB // EB
ROWS_PER_SUB = N_NODES // NS
ZB = 125


def _edge_linear_body(ea_ref, w_ref, b_ref, out_ref):
    out_ref[...] = (
        jnp.dot(ea_ref[...], w_ref[...], preferred_element_type=jnp.float32)
        + b_ref[...]
    )


def _edge_linear(edge_attr, W_edge, b2d):
    BE = 2000
    grid = (NC, N_EDGES // BE)
    return pl.pallas_call(
        _edge_linear_body,
        grid=grid,
        in_specs=[
            pl.BlockSpec((BE, ED), lambda c, i: (i, 0)),
            pl.BlockSpec((ED, HALF), lambda c, i: (0, c)),
            pl.BlockSpec((1, HALF), lambda c, i: (c, 0)),
        ],
        out_specs=pl.BlockSpec((BE, HALF), lambda c, i: (c * (N_EDGES // BE) + i, 0)),
        out_shape=jax.ShapeDtypeStruct((NC * N_EDGES, HALF), jnp.float32),
    )(edge_attr, W_edge, b2d)


def _sc_message_body(x2, e2, src, dst, acc_out, idx_d, idx_s, xbuf, ebuf,
                     zbuf, acc_sh, sem):
    c = lax.axis_index("c")
    s = lax.axis_index("s")

    def zfill(i, _):
        zbuf[i, :] = jnp.zeros((16,), jnp.float32)
        return 0
    lax.fori_loop(0, ZB * HALF // 16, zfill, 0)
    zview = zbuf.reshape(ZB, HALF)
    for k in range(ROWS_PER_SUB // ZB):
        pltpu.sync_copy(zview, acc_sh.at[pl.ds(s * ROWS_PER_SUB + k * ZB, ZB)])
    plsc.subcore_barrier()

    cN = (c * N_NODES).astype(jnp.int32)

    def block(bi, _):
        base = s * EDGES_PER_SUB + bi * EB
        pltpu.sync_copy(dst.at[pl.ds(base, EB)], idx_d)
        pltpu.sync_copy(src.at[pl.ds(base, EB)], idx_s)

        def addoff(i, _):
            idx_d[pl.ds(i * 16, 16)] = idx_d[pl.ds(i * 16, 16)] + cN
            return 0
        lax.fori_loop(0, EB // 16, addoff, 0)

        gat = pltpu.async_copy(x2.at[idx_d], xbuf, sem)
        pltpu.sync_copy(e2.at[pl.ds(c * N_EDGES + base, EB)], ebuf)
        gat.wait()

        def row(r, _):
            for v in range(HALF // 16):
                sl = pl.ds(v * 16, 16)
                ebuf[r, sl] = jnp.maximum(xbuf[r, sl] + ebuf[r, sl], 0.0)
            return 0
        lax.fori_loop(0, EB, row, 0)

        pltpu.sync_copy(ebuf, acc_sh.at[idx_s], add=True)
        return 0

    lax.fori_loop(0, N_BLOCKS, block, 0)
    plsc.subcore_barrier()

    for k in range(ROWS_PER_SUB // ZB):
        row0 = s * ROWS_PER_SUB + k * ZB
        pltpu.sync_copy(acc_sh.at[pl.ds(row0, ZB)], zview)
        pltpu.sync_copy(zview, acc_out.at[pl.ds(c * N_NODES + row0, ZB)])


def _sc_message(x2, e2, src, dst):
    mesh = plsc.VectorSubcoreMesh(core_axis_name="c", subcore_axis_name="s")
    return pl.kernel(
        _sc_message_body,
        out_type=jax.ShapeDtypeStruct((NC * N_NODES, HALF), jnp.float32),
        mesh=mesh,
        scratch_types=[
            pltpu.VMEM((EB,), jnp.int32),
            pltpu.VMEM((EB,), jnp.int32),
            pltpu.VMEM((EB, HALF), jnp.float32),
            pltpu.VMEM((EB, HALF), jnp.float32),
            pltpu.VMEM((ZB * HALF // 16, 16), jnp.float32),
            pltpu.VMEM_SHARED((N_NODES, HALF), jnp.float32),
            pltpu.SemaphoreType.DMA,
        ],
    )(x2, e2, src, dst)


def _mlp_body(eps_ref, x_ref, a0_ref, a1_ref, w1_ref, b1_ref, g_ref, be_ref,
              w2_ref, b2_ref, out_ref):
    h = (1.0 + eps_ref[0]) * x_ref[...] + jnp.concatenate(
        [a0_ref[0], a1_ref[0]], axis=1)
    h1 = jnp.dot(h, w1_ref[...], preferred_element_type=jnp.float32) + b1_ref[...]
    mu = jnp.mean(h1, axis=-1, keepdims=True)
    var = jnp.mean(jnp.square(h1 - mu), axis=-1, keepdims=True)
    h1n = (h1 - mu) * lax.rsqrt(var + 1e-5) * g_ref[...] + be_ref[...]
    out_ref[...] = (
        jnp.dot(jnp.maximum(h1n, 0.0), w2_ref[...],
                preferred_element_type=jnp.float32)
        + b2_ref[...]
    )


def _mlp(eps, x, acc3, W1, b1, ln_gamma, ln_beta, W2, b2):
    BN = 1000
    grid = (N_NODES // BN,)
    return pl.pallas_call(
        _mlp_body,
        grid=grid,
        in_specs=[
            pl.BlockSpec(memory_space=pltpu.SMEM),
            pl.BlockSpec((BN, D), lambda i: (i, 0)),
            pl.BlockSpec((1, BN, HALF), lambda i: (0, i, 0)),
            pl.BlockSpec((1, BN, HALF), lambda i: (1, i, 0)),
            pl.BlockSpec((D, 2 * D), lambda i: (0, 0)),
            pl.BlockSpec((1, 2 * D), lambda i: (0, 0)),
            pl.BlockSpec((1, 2 * D), lambda i: (0, 0)),
            pl.BlockSpec((1, 2 * D), lambda i: (0, 0)),
            pl.BlockSpec((2 * D, D), lambda i: (0, 0)),
            pl.BlockSpec((1, D), lambda i: (0, 0)),
        ],
        out_specs=pl.BlockSpec((BN, D), lambda i: (i, 0)),
        out_shape=jax.ShapeDtypeStruct((N_NODES, D), jnp.float32),
    )(eps, x, acc3, acc3, W1, b1.reshape(1, -1), ln_gamma.reshape(1, -1),
      ln_beta.reshape(1, -1), W2, b2.reshape(1, -1))


@jax.jit
def kernel(x, edge_index, edge_attr, eps, W_edge, b_edge, W1, b1, ln_gamma,
           ln_beta, W2, b2):
    src = edge_index[0].astype(jnp.int32)
    dst = edge_index[1].astype(jnp.int32)

    e2 = _edge_linear(edge_attr, W_edge, b_edge.reshape(NC, HALF))
    x2 = x.reshape(N_NODES, NC, HALF).transpose(1, 0, 2).reshape(
        NC * N_NODES, HALF)

    acc = _sc_message(x2, e2, src, dst)
    acc3 = acc.reshape(NC, N_NODES, HALF)
    return _mlp(eps, x, acc3, W1, b1, ln_gamma, ln_beta, W2, b2)


# speedup vs baseline: 1.9461x; 1.9461x over previous
import functools

import jax
import jax.numpy as jnp
from jax import lax
from jax.experimental import pallas as pl
from jax.experimental.pallas import tpu as pltpu
from jax.experimental.pallas import tpu_sc as plsc

N_NODES = 10000
N_EDGES = 160000
D = 256
ED = 16
HALF = D // 2

NC = 2
NS = 16
EDGES_PER_SUB = N_EDGES // NS
EB = 80
N_BLOCKS = EDGES_PER_SUB // EB
ZB = 80
N_CHUNKS = N_NODES // ZB


def _edge_linear_body(ea_ref, w_ref, b_ref, out_ref):
    out_ref[...] = (
        jnp.dot(ea_ref[...], w_ref[...], preferred_element_type=jnp.float32)
        + b_ref[0, 0:1, :]
    )


def _edge_linear(edge_attr, W_edge, b2d):
    BE = 2000
    grid = (NC, N_EDGES // BE)
    return pl.pallas_call(
        _edge_linear_body,
        grid=grid,
        in_specs=[
            pl.BlockSpec((BE, ED), lambda c, i: (i, 0)),
            pl.BlockSpec((ED, HALF), lambda c, i: (0, c)),
            pl.BlockSpec((1, 8, HALF), lambda c, i: (c, 0, 0)),
        ],
        out_specs=pl.BlockSpec((BE, HALF), lambda c, i: (c * (N_EDGES // BE) + i, 0)),
        out_shape=jax.ShapeDtypeStruct((NC * N_EDGES, HALF), jnp.float32),
    )(edge_attr, W_edge, b2d)


def _sc_message_body(x2, e2, src, dst, acc_out, idx_d, idx_s, xbuf, ebuf,
                     acc_sh, sem):
    c = lax.axis_index("c")
    s = lax.axis_index("s")

    def zfill(r, _):
        for v in range(HALF // 16):
            ebuf[r, pl.ds(v * 16, 16)] = jnp.zeros((16,), jnp.float32)
        return 0
    lax.fori_loop(0, ZB, zfill, 0)
    for k in range(N_CHUNKS):
        @pl.when(s == k % NS)
        def _():
            pltpu.sync_copy(ebuf, acc_sh.at[pl.ds(k * ZB, ZB)])
    plsc.subcore_barrier()

    cN = (c * N_NODES).astype(jnp.int32)

    def block(bi, _):
        base = s * EDGES_PER_SUB + bi * EB
        pltpu.sync_copy(dst.at[pl.ds(base, EB)], idx_d)
        pltpu.sync_copy(src.at[pl.ds(base, EB)], idx_s)

        def addoff(i, _):
            idx_d[pl.ds(i * 16, 16)] = idx_d[pl.ds(i * 16, 16)] + cN
            return 0
        lax.fori_loop(0, EB // 16, addoff, 0)

        gat = pltpu.async_copy(x2.at[idx_d], xbuf, sem)
        pltpu.sync_copy(e2.at[pl.ds(c * N_EDGES + base, EB)], ebuf)
        gat.wait()

        def row(r, _):
            for v in range(HALF // 16):
                sl = pl.ds(v * 16, 16)
                ebuf[r, sl] = jnp.maximum(xbuf[r, sl] + ebuf[r, sl], 0.0)
            return 0
        lax.fori_loop(0, EB, row, 0)

        pltpu.sync_copy(ebuf, acc_sh.at[idx_s], add=True)
        return 0

    lax.fori_loop(0, N_BLOCKS, block, 0)
    plsc.subcore_barrier()

    for k in range(N_CHUNKS):
        @pl.when(s == k % NS)
        def _():
            pltpu.sync_copy(acc_sh.at[pl.ds(k * ZB, ZB)], ebuf)
            pltpu.sync_copy(ebuf, acc_out.at[pl.ds(c * N_NODES + k * ZB, ZB)])


def _sc_message(x2, e2, src, dst):
    mesh = plsc.VectorSubcoreMesh(core_axis_name="c", subcore_axis_name="s")
    return pl.kernel(
        _sc_message_body,
        out_type=jax.ShapeDtypeStruct((NC * N_NODES, HALF), jnp.float32),
        mesh=mesh,
        scratch_types=[
            pltpu.VMEM((EB,), jnp.int32),
            pltpu.VMEM((EB,), jnp.int32),
            pltpu.VMEM((EB, HALF), jnp.float32),
            pltpu.VMEM((EB, HALF), jnp.float32),
            pltpu.VMEM_SHARED((N_NODES, HALF), jnp.float32),
            pltpu.SemaphoreType.DMA,
        ],
    )(x2, e2, src, dst)


def _mlp_body(eps_ref, x_ref, a0_ref, a1_ref, w1_ref, b1_ref, g_ref, be_ref,
              w2_ref, b2_ref, out_ref):
    h = (1.0 + eps_ref[0]) * x_ref[...] + jnp.concatenate(
        [a0_ref[0], a1_ref[0]], axis=1)
    h1 = jnp.dot(h, w1_ref[...], preferred_element_type=jnp.float32) + b1_ref[...]
    mu = jnp.mean(h1, axis=-1, keepdims=True)
    var = jnp.mean(jnp.square(h1 - mu), axis=-1, keepdims=True)
    h1n = (h1 - mu) * lax.rsqrt(var + 1e-5) * g_ref[...] + be_ref[...]
    out_ref[...] = (
        jnp.dot(jnp.maximum(h1n, 0.0), w2_ref[...],
                preferred_element_type=jnp.float32)
        + b2_ref[...]
    )


def _mlp(eps, x, acc3, W1, b1, ln_gamma, ln_beta, W2, b2):
    BN = 1000
    grid = (N_NODES // BN,)
    return pl.pallas_call(
        _mlp_body,
        grid=grid,
        in_specs=[
            pl.BlockSpec(memory_space=pltpu.SMEM),
            pl.BlockSpec((BN, D), lambda i: (i, 0)),
            pl.BlockSpec((1, BN, HALF), lambda i: (0, i, 0)),
            pl.BlockSpec((1, BN, HALF), lambda i: (1, i, 0)),
            pl.BlockSpec((D, 2 * D), lambda i: (0, 0)),
            pl.BlockSpec((1, 2 * D), lambda i: (0, 0)),
            pl.BlockSpec((1, 2 * D), lambda i: (0, 0)),
            pl.BlockSpec((1, 2 * D), lambda i: (0, 0)),
            pl.BlockSpec((2 * D, D), lambda i: (0, 0)),
            pl.BlockSpec((1, D), lambda i: (0, 0)),
        ],
        out_specs=pl.BlockSpec((BN, D), lambda i: (i, 0)),
        out_shape=jax.ShapeDtypeStruct((N_NODES, D), jnp.float32),
    )(eps, x, acc3, acc3, W1, b1.reshape(1, -1), ln_gamma.reshape(1, -1),
      ln_beta.reshape(1, -1), W2, b2.reshape(1, -1))


@jax.jit
def kernel(x, edge_index, edge_attr, eps, W_edge, b_edge, W1, b1, ln_gamma,
           ln_beta, W2, b2):
    src = edge_index[0].astype(jnp.int32)
    dst = edge_index[1].astype(jnp.int32)

    b2d = jnp.broadcast_to(b_edge.reshape(NC, 1, HALF), (NC, 8, HALF))
    e2 = _edge_linear(edge_attr, W_edge, b2d)
    x2 = x.reshape(N_NODES, NC, HALF).transpose(1, 0, 2).reshape(
        NC * N_NODES, HALF)

    acc = _sc_message(x2, e2, src, dst)
    acc3 = acc.reshape(NC, N_NODES, HALF)
    return _mlp(eps, x, acc3, W1, b1, ln_gamma, ln_beta, W2, b2)
